# Initial kernel scaffold; baseline (speedup 1.0000x reference)
import jax
import jax.numpy as jnp
from jax.experimental import pallas as pl


def kernel(inputs, codebooks):
    raise NotImplementedError("write your pallas kernel here")


# speedup vs baseline: 1.2732x; 1.2732x over previous
import jax
import jax.numpy as jnp
from jax.experimental import pallas as pl
from jax.experimental.pallas import tpu as pltpu

NUM_SECTIONS = 4
NUM_CENTROIDS = 1024
EMBED_DIM = 256
SEC_DIM = EMBED_DIM // NUM_SECTIONS
COMMITMENT = 0.25
TM = 512


def _pq_kernel(x_ref, cb_ref, q_ref, idx_ref, loss_ref):
    i = pl.program_id(0)

    @pl.when(i == 0)
    def _():
        loss_ref[0, 0] = 0.0

    x_all = x_ref[...]
    scale = (1.0 + COMMITMENT) / (NUM_SECTIONS * SEC_DIM)
    qs = []
    for s in range(NUM_SECTIONS):
        x = x_all[:, s * SEC_DIM:(s + 1) * SEC_DIM]
        cb = cb_ref[s]
        scores = jax.lax.dot_general(
            x, cb, dimension_numbers=(((1,), (1,)), ((), ())),
            preferred_element_type=jnp.float32)
        xn = jnp.sum(x * x, axis=1, keepdims=True)
        cn = jnp.sum(cb * cb, axis=1)
        d = (xn - 2.0 * scores) + cn[None, :]

        dmin = jnp.min(d, axis=1, keepdims=True)
        iota = jax.lax.broadcasted_iota(jnp.int32, d.shape, 1)
        idx = jnp.min(jnp.where(d == dmin, iota, NUM_CENTROIDS), axis=1)
        idx_ref[s, 0, :] = idx

        onehot = (iota == idx[:, None]).astype(jnp.float32)
        q = jax.lax.dot_general(
            onehot, cb, dimension_numbers=(((1,), (0,)), ((), ())),
            preferred_element_type=jnp.float32,
            precision=jax.lax.Precision.HIGHEST)
        qs.append(q)
        loss_ref[0, 0] += jnp.sum((q - x) ** 2) * scale

    q_ref[...] = jnp.concatenate(qs, axis=1)


def kernel(inputs, codebooks):
    B, T, _ = inputs.shape
    N = B * T
    nblk = N // TM
    x2d = inputs.reshape(N, EMBED_DIM)

    q2d, idx3d, loss = pl.pallas_call(
        _pq_kernel,
        grid=(nblk,),
        in_specs=[
            pl.BlockSpec((TM, EMBED_DIM), lambda i: (i, 0)),
            pl.BlockSpec((NUM_SECTIONS, NUM_CENTROIDS, SEC_DIM),
                         lambda i: (0, 0, 0)),
        ],
        out_specs=[
            pl.BlockSpec((TM, EMBED_DIM), lambda i: (i, 0)),
            pl.BlockSpec((NUM_SECTIONS, 1, TM), lambda i: (i, 0, 0)),
            pl.BlockSpec(memory_space=pltpu.SMEM),
        ],
        out_shape=[
            jax.ShapeDtypeStruct((N, EMBED_DIM), jnp.float32),
            jax.ShapeDtypeStruct((nblk * NUM_SECTIONS, 1, TM), jnp.int32),
            jax.ShapeDtypeStruct((1, 1), jnp.float32),
        ],
    )(x2d, codebooks)

    quantized = q2d.reshape(B, T, EMBED_DIM)
    nn_idx = (idx3d.reshape(nblk, NUM_SECTIONS, TM)
              .transpose(1, 0, 2).reshape(NUM_SECTIONS, B, T))
    codebook = codebooks.reshape(NUM_SECTIONS * NUM_CENTROIDS, SEC_DIM)
    return (quantized, loss[0, 0] / N, nn_idx, codebook)


# speedup vs baseline: 1.9266x; 1.5132x over previous
import functools

import jax
import jax.numpy as jnp
from jax import lax
from jax.experimental import pallas as pl
from jax.experimental.pallas import tpu as pltpu
from jax.experimental.pallas import tpu_sc as plsc

NUM_SECTIONS = 4
NUM_CENTROIDS = 1024
EMBED_DIM = 256
SEC_DIM = EMBED_DIM // NUM_SECTIONS
COMMITMENT = 0.25
TM = 576
NW = 32
BLK_PER_W = 2


def _dist_kernel(x_ref, cb_ref, idx_ref, idxg_ref, loss_ref):
    i = pl.program_id(0)

    @pl.when(i == 0)
    def _():
        loss_ref[0, 0] = 0.0

    x_all = x_ref[...]
    scale = (1.0 + COMMITMENT) / (NUM_SECTIONS * SEC_DIM)
    for s in range(NUM_SECTIONS):
        x = x_all[:, s * SEC_DIM:(s + 1) * SEC_DIM]
        cb = cb_ref[s]
        scores = jax.lax.dot_general(
            x, cb, dimension_numbers=(((1,), (1,)), ((), ())),
            preferred_element_type=jnp.float32)
        xn = jnp.sum(x * x, axis=1, keepdims=True)
        cn = jnp.sum(cb * cb, axis=1)
        d = (xn - 2.0 * scores) + cn[None, :]

        dmin = jnp.min(d, axis=1, keepdims=True)
        fiota = jax.lax.broadcasted_iota(
            jnp.int32, d.shape, 1).astype(jnp.float32)
        idx_f = jnp.min(jnp.where(d == dmin, fiota, float(NUM_CENTROIDS)),
                        axis=1)
        idx = idx_f.astype(jnp.int32)
        idx_ref[s, 0, :] = idx
        idxg_ref[s, 0, :] = idx + s * NUM_CENTROIDS
        loss_ref[0, 0] += jnp.sum(dmin) * scale


def _gather_body(table_hbm, idxg_hbm, out_hbm, idx_v, rows_v, sem):
    wid = lax.axis_index("s") * 2 + lax.axis_index("c")
    for b in range(BLK_PER_W):
        blk = wid * BLK_PER_W + b
        for s in range(NUM_SECTIONS):
            pltpu.sync_copy(idxg_hbm.at[blk * NUM_SECTIONS + s, 0], idx_v)
            pltpu.async_copy(table_hbm.at[idx_v], rows_v, sem).wait()
            pltpu.sync_copy(
                rows_v,
                out_hbm.at[pl.ds(blk * TM, TM),
                           pl.ds(s * SEC_DIM, SEC_DIM)])


def kernel(inputs, codebooks):
    B, T, _ = inputs.shape
    N = B * T
    nblk = N // TM
    x2d = inputs.reshape(N, EMBED_DIM)

    idx3d, idxg3d, loss = pl.pallas_call(
        _dist_kernel,
        grid=(nblk,),
        in_specs=[
            pl.BlockSpec((TM, EMBED_DIM), lambda i: (i, 0)),
            pl.BlockSpec((NUM_SECTIONS, NUM_CENTROIDS, SEC_DIM),
                         lambda i: (0, 0, 0)),
        ],
        out_specs=[
            pl.BlockSpec((NUM_SECTIONS, 1, TM), lambda i: (i, 0, 0)),
            pl.BlockSpec((NUM_SECTIONS, 1, TM), lambda i: (i, 0, 0)),
            pl.BlockSpec(memory_space=pltpu.SMEM),
        ],
        out_shape=[
            jax.ShapeDtypeStruct((nblk * NUM_SECTIONS, 1, TM), jnp.int32),
            jax.ShapeDtypeStruct((nblk * NUM_SECTIONS, 1, TM), jnp.int32),
            jax.ShapeDtypeStruct((1, 1), jnp.float32),
        ],
    )(x2d, codebooks)

    table = codebooks.reshape(NUM_SECTIONS * NUM_CENTROIDS, SEC_DIM)

    gather = pl.kernel(
        _gather_body,
        out_type=jax.ShapeDtypeStruct((N, EMBED_DIM), jnp.float32),
        mesh=plsc.VectorSubcoreMesh(core_axis_name="c", subcore_axis_name="s"),
        compiler_params=pltpu.CompilerParams(use_tc_tiling_on_sc=False),
        scratch_types=[
            pltpu.VMEM((TM,), jnp.int32),
            pltpu.VMEM((TM, SEC_DIM), jnp.float32),
            pltpu.SemaphoreType.DMA,
        ],
    )
    q2d = gather(table, idxg3d)

    quantized = q2d.reshape(B, T, EMBED_DIM)
    nn_idx = (idx3d.reshape(nblk, NUM_SECTIONS, TM)
              .transpose(1, 0, 2).reshape(NUM_SECTIONS, B, T))
    return (quantized, loss[0, 0] / N, nn_idx, table)


# speedup vs baseline: 2.8361x; 1.4721x over previous
import jax
import jax.numpy as jnp
from jax import lax
from jax.experimental import pallas as pl
from jax.experimental.pallas import tpu as pltpu
from jax.experimental.pallas import tpu_sc as plsc

NUM_SECTIONS = 4
NUM_CENTROIDS = 1024
EMBED_DIM = 256
SEC_DIM = EMBED_DIM // NUM_SECTIONS
COMMITMENT = 0.25
TM = 512
NW = 32
KAUG = 2 * SEC_DIM


def _dist_kernel(x_ref, cbm2_ref, cnb_ref, fiota_ref, idxg_ref, loss_ref):
    i = pl.program_id(0)

    @pl.when(i == 0)
    def _():
        loss_ref[0, 0] = 0.0

    x_all = x_ref[...]
    fiota = fiota_ref[...]
    scale = (1.0 + COMMITMENT) / (NUM_SECTIONS * SEC_DIM)
    loss_step = jnp.sum(x_all * x_all)
    for s in range(NUM_SECTIONS):
        x = x_all[:, s * SEC_DIM:(s + 1) * SEC_DIM]
        dT = jax.lax.dot_general(
            cbm2_ref[s], x, dimension_numbers=(((1,), (1,)), ((), ())),
            preferred_element_type=jnp.float32) + cnb_ref[s]
        dmin = jnp.min(dT, axis=0, keepdims=True)
        sel = jnp.where(dT == dmin, fiota, 2048.0)
        idxf = jnp.min(sel, axis=0)
        idxg_ref[s, :] = idxf.astype(jnp.int32) + s * NUM_CENTROIDS
        loss_step += jnp.sum(dmin)
    loss_ref[0, 0] += loss_step * scale


def _gather_body(table_hbm, idxg_hbm, out_hbm, idx_v, rows_v, sem):
    wid = lax.axis_index("s") * 2 + lax.axis_index("c")
    tpw = (64 * 576) // NW
    base = wid * tpw
    for s in range(NUM_SECTIONS):
        pltpu.sync_copy(idxg_hbm.at[s, pl.ds(base, tpw)], idx_v)
        pltpu.async_copy(table_hbm.at[idx_v], rows_v, sem).wait()
        pltpu.sync_copy(
            rows_v,
            out_hbm.at[pl.ds(base, tpw), pl.ds(s * SEC_DIM, SEC_DIM)])


def kernel(inputs, codebooks):
    B, T, _ = inputs.shape
    N = B * T
    nblk = N // TM
    x2d = inputs.reshape(N, EMBED_DIM)

    cbm2 = -2.0 * codebooks
    cn = jnp.sum(codebooks * codebooks, axis=2)
    cnb = jnp.broadcast_to(cn[:, :, None],
                           (NUM_SECTIONS, NUM_CENTROIDS, TM))
    fiota = jax.lax.broadcasted_iota(jnp.float32, (NUM_CENTROIDS, TM), 0)

    idxg, loss = pl.pallas_call(
        _dist_kernel,
        grid=(nblk,),
        in_specs=[
            pl.BlockSpec((TM, EMBED_DIM), lambda i: (i, 0)),
            pl.BlockSpec((NUM_SECTIONS, NUM_CENTROIDS, SEC_DIM),
                         lambda i: (0, 0, 0)),
            pl.BlockSpec((NUM_SECTIONS, NUM_CENTROIDS, TM),
                         lambda i: (0, 0, 0)),
            pl.BlockSpec((NUM_CENTROIDS, TM), lambda i: (0, 0)),
        ],
        out_specs=[
            pl.BlockSpec((NUM_SECTIONS, TM), lambda i: (0, i)),
            pl.BlockSpec(memory_space=pltpu.SMEM),
        ],
        out_shape=[
            jax.ShapeDtypeStruct((NUM_SECTIONS, N), jnp.int32),
            jax.ShapeDtypeStruct((1, 1), jnp.float32),
        ],
    )(x2d, cbm2, cnb, fiota)

    table = codebooks.reshape(NUM_SECTIONS * NUM_CENTROIDS, SEC_DIM)

    gather = pl.kernel(
        _gather_body,
        out_type=jax.ShapeDtypeStruct((N, EMBED_DIM), jnp.float32),
        mesh=plsc.VectorSubcoreMesh(core_axis_name="c", subcore_axis_name="s"),
        compiler_params=pltpu.CompilerParams(use_tc_tiling_on_sc=False),
        scratch_types=[
            pltpu.VMEM((N // NW,), jnp.int32),
            pltpu.VMEM((N // NW, SEC_DIM), jnp.float32),
            pltpu.SemaphoreType.DMA,
        ],
    )
    q2d = gather(table, idxg)

    quantized = q2d.reshape(B, T, EMBED_DIM)
    offs = (jnp.arange(NUM_SECTIONS, dtype=jnp.int32)
            * NUM_CENTROIDS)[:, None]
    nn_idx = (idxg - offs).reshape(NUM_SECTIONS, B, T)
    return (quantized, loss[0, 0] / N, nn_idx, table)
